# baseline (speedup 1.0000x reference)
import functools
import jax
import jax.numpy as jnp
from jax import lax
from jax.experimental import pallas as pl
from jax.experimental.pallas import tpu as pltpu
from jax.experimental.pallas import tpu_sc as plsc

_B, _S, _D = 4, 8192, 1024
_NW = 32
_TOT = _B * _S * _D
_EMB_TOT = _S * _D
_PER_W = _TOT // _NW
_CHUNK = 32768
_STEPS = _PER_W // _CHUNK


@functools.partial(
    pl.kernel,
    mesh=plsc.VectorSubcoreMesh(core_axis_name="c", subcore_axis_name="s"),
    out_type=jax.ShapeDtypeStruct((_TOT,), jnp.float32),
    scratch_types=[
        pltpu.VMEM((_CHUNK,), jnp.float32),
        pltpu.VMEM((_CHUNK,), jnp.float32),
    ],
)
def _sc_add(x_hbm, emb_hbm, out_hbm, xbuf, ebuf):
    wid = lax.axis_index("s") * 2 + lax.axis_index("c")
    base = wid * _PER_W
    ebase = lax.rem(base, _EMB_TOT)

    def step(t, _):
        off = base + t * _CHUNK
        eoff = ebase + t * _CHUNK
        pltpu.sync_copy(x_hbm.at[pl.ds(off, _CHUNK)], xbuf)
        pltpu.sync_copy(emb_hbm.at[pl.ds(eoff, _CHUNK)], ebuf)

        def add16(i, _):
            sl = pl.ds(i * 16, 16)
            xbuf[sl] = xbuf[sl] + ebuf[sl]
            return 0

        lax.fori_loop(0, _CHUNK // 16, add16, 0)
        pltpu.sync_copy(xbuf, out_hbm.at[pl.ds(off, _CHUNK)])
        return 0

    lax.fori_loop(0, _STEPS, step, 0)


def kernel(x, emb):
    out = _sc_add(x.reshape(-1), emb.reshape(-1))
    return out.reshape(x.shape)


# speedup vs baseline: 1.5629x; 1.5629x over previous
import functools
import jax
import jax.numpy as jnp
from jax import lax
from jax.experimental import pallas as pl
from jax.experimental.pallas import tpu as pltpu
from jax.experimental.pallas import tpu_sc as plsc

_B, _S, _D = 4, 8192, 1024
_NW = 32
_TOT = _B * _S * _D
_EMB_TOT = _S * _D
_PER_W = _TOT // _NW
_CHUNK = 16384
_STEPS = _PER_W // _CHUNK
_UNROLL = 8


@functools.partial(
    pl.kernel,
    mesh=plsc.VectorSubcoreMesh(core_axis_name="c", subcore_axis_name="s"),
    out_type=jax.ShapeDtypeStruct((_TOT,), jnp.float32),
    scratch_types=[
        pltpu.VMEM((2, _CHUNK), jnp.float32),
        pltpu.VMEM((2, _CHUNK), jnp.float32),
        pltpu.VMEM((2, _CHUNK), jnp.float32),
        pltpu.SemaphoreType.DMA,
        pltpu.SemaphoreType.DMA,
        pltpu.SemaphoreType.DMA,
        pltpu.SemaphoreType.DMA,
        pltpu.SemaphoreType.DMA,
        pltpu.SemaphoreType.DMA,
    ],
)
def _sc_add(x_hbm, emb_hbm, out_hbm, xbuf, ebuf, obuf,
            sx0, sx1, se0, se1, so0, so1):
    sx, se, so = [sx0, sx1], [se0, se1], [so0, so1]
    wid = lax.axis_index("s") * 2 + lax.axis_index("c")
    base = wid * _PER_W
    ebase = lax.rem(base, _EMB_TOT)

    def x_slice(t):
        return x_hbm.at[pl.ds(base + t * _CHUNK, _CHUNK)]

    def e_slice(t):
        return emb_hbm.at[pl.ds(ebase + t * _CHUNK, _CHUNK)]

    def o_slice(t):
        return out_hbm.at[pl.ds(base + t * _CHUNK, _CHUNK)]

    for b in range(2):
        pltpu.async_copy(x_slice(b), xbuf.at[b], sx[b])
        pltpu.async_copy(e_slice(b), ebuf.at[b], se[b])

    def outer(g, _):
        for b in range(2):
            t = 2 * g + b
            pltpu.make_async_copy(x_slice(t), xbuf.at[b], sx[b]).wait()
            pltpu.make_async_copy(e_slice(t), ebuf.at[b], se[b]).wait()

            @pl.when(g > 0)
            def _wait_store():
                pltpu.make_async_copy(obuf.at[b], o_slice(t - 2), so[b]).wait()

            def add_u(i, _):
                for u in range(_UNROLL):
                    sl = pl.ds((i * _UNROLL + u) * 16, 16)
                    obuf[b, sl] = xbuf[b, sl] + ebuf[b, sl]
                return 0

            lax.fori_loop(0, _CHUNK // (16 * _UNROLL), add_u, 0)

            pltpu.async_copy(obuf.at[b], o_slice(t), so[b])

            @pl.when(g < _STEPS // 2 - 1)
            def _next_loads():
                pltpu.async_copy(x_slice(t + 2), xbuf.at[b], sx[b])
                pltpu.async_copy(e_slice(t + 2), ebuf.at[b], se[b])

        return 0

    lax.fori_loop(0, _STEPS // 2, outer, 0)

    for b in range(2):
        pltpu.make_async_copy(obuf.at[b], o_slice(_STEPS - 2 + b), so[b]).wait()


def kernel(x, emb):
    out = _sc_add(x.reshape(-1), emb.reshape(-1))
    return out.reshape(x.shape)


# speedup vs baseline: 3.6556x; 2.3389x over previous
import functools
import jax
import jax.numpy as jnp
from jax import lax
from jax.experimental import pallas as pl
from jax.experimental.pallas import tpu as pltpu
from jax.experimental.pallas import tpu_sc as plsc

_B, _S, _D = 4, 8192, 1024
_NW = 32
_ROWS = _B * _S
_PER_W = _ROWS // _NW
_T = 16
_STEPS = _PER_W // _T


@functools.partial(
    pl.kernel,
    mesh=plsc.VectorSubcoreMesh(core_axis_name="c", subcore_axis_name="s"),
    out_type=jax.ShapeDtypeStruct((_ROWS, _D), jnp.float32),
    scratch_types=[
        pltpu.VMEM((2, _T, _D), jnp.float32),
        pltpu.VMEM((2, _T, _D), jnp.float32),
        pltpu.VMEM((2, _T, _D), jnp.float32),
        pltpu.SemaphoreType.DMA,
        pltpu.SemaphoreType.DMA,
        pltpu.SemaphoreType.DMA,
        pltpu.SemaphoreType.DMA,
        pltpu.SemaphoreType.DMA,
        pltpu.SemaphoreType.DMA,
    ],
)
def _sc_add(x_hbm, emb_hbm, out_hbm, xbuf, ebuf, obuf,
            sx0, sx1, se0, se1, so0, so1):
    sx, se, so = [sx0, sx1], [se0, se1], [so0, so1]
    wid = lax.axis_index("s") * 2 + lax.axis_index("c")
    base = wid * _PER_W
    ebase = lax.rem(base, _S)

    def x_slice(t):
        return x_hbm.at[pl.ds(base + t * _T, _T), :]

    def e_slice(t):
        return emb_hbm.at[pl.ds(ebase + t * _T, _T), :]

    def o_slice(t):
        return out_hbm.at[pl.ds(base + t * _T, _T), :]

    for b in range(2):
        pltpu.async_copy(x_slice(b), xbuf.at[b], sx[b])
        pltpu.async_copy(e_slice(b), ebuf.at[b], se[b])

    def outer(g, _):
        for b in range(2):
            t = 2 * g + b
            pltpu.make_async_copy(x_slice(t), xbuf.at[b], sx[b]).wait()
            pltpu.make_async_copy(e_slice(t), ebuf.at[b], se[b]).wait()

            @pl.when(g > 0)
            def _wait_store():
                pltpu.make_async_copy(obuf.at[b], o_slice(t - 2), so[b]).wait()

            def add_row(r, _):
                for u in range(_D // 16):
                    sl = pl.ds(u * 16, 16)
                    obuf[b, r, sl] = xbuf[b, r, sl] + ebuf[b, r, sl]
                return 0

            lax.fori_loop(0, _T, add_row, 0)

            pltpu.async_copy(obuf.at[b], o_slice(t), so[b])

            @pl.when(g < _STEPS // 2 - 1)
            def _next_loads():
                pltpu.async_copy(x_slice(t + 2), xbuf.at[b], sx[b])
                pltpu.async_copy(e_slice(t + 2), ebuf.at[b], se[b])

        return 0

    lax.fori_loop(0, _STEPS // 2, outer, 0)

    for b in range(2):
        pltpu.make_async_copy(obuf.at[b], o_slice(_STEPS - 2 + b), so[b]).wait()


def kernel(x, emb):
    B, S, D = x.shape
    out = _sc_add(x.reshape(B * S, D), emb)
    return out.reshape(B, S, D)
